# baseline (speedup 1.0000x reference)
import functools

import jax
import jax.numpy as jnp
from jax import lax
from jax.experimental import pallas as pl
from jax.experimental.pallas import tpu as pltpu
from jax.experimental.pallas import tpu_sc as plsc

NC = 2
NS = 16
NW = NC * NS
CH = 80
NB = 2


def _mesh():
    return plsc.VectorSubcoreMesh(
        core_axis_name="c", subcore_axis_name="s", num_cores=NC,
        num_subcores=NS)


def _zero_spmem(s, zb_hbm, acc_sh, n, sem):
    n_zchunks = n // CH
    n_iter = (n_zchunks + NS - 1) // NS
    def fire(j, carry):
        k = s + j * NS
        @pl.when(k < n_zchunks)
        def _():
            pltpu.async_copy(zb_hbm, acc_sh.at[pl.ds(k * CH, CH)], sem)
        return carry
    lax.fori_loop(0, n_iter, fire, 0)
    def drain(j, carry):
        k = s + j * NS
        @pl.when(k < n_zchunks)
        def _():
            pltpu.make_async_copy(zb_hbm, acc_sh.at[pl.ds(k * CH, CH)],
                                  sem).wait()
        return carry
    lax.fori_loop(0, n_iter, drain, 0)


def _writeback(s, c, acc_sh, out_hbm, n, sem):
    n_zchunks = n // CH
    n_iter = (n_zchunks + NS - 1) // NS
    def fire(j, carry):
        k = s + j * NS
        @pl.when(k < n_zchunks)
        def _():
            pltpu.async_copy(acc_sh.at[pl.ds(k * CH, CH)],
                             out_hbm.at[pl.ds(c * n + k * CH, CH)], sem)
        return carry
    lax.fori_loop(0, n_iter, fire, 0)
    def drain(j, carry):
        k = s + j * NS
        @pl.when(k < n_zchunks)
        def _():
            pltpu.make_async_copy(acc_sh.at[pl.ds(k * CH, CH)],
                                  out_hbm.at[pl.ds(c * n + k * CH, CH)],
                                  sem).wait()
        return carry
    lax.fori_loop(0, n_iter, drain, 0)


def _sc_gather_pass(x, row4, col4, n, e, d_node):
    n_chunks = e // (NW * CH)
    assert e == n_chunks * NW * CH and n % CH == 0 and n_chunks > NB

    @functools.partial(
        pl.kernel,
        out_type=jax.ShapeDtypeStruct((NC * n, d_node), jnp.float32),
        mesh=_mesh(),
        scratch_types=[
            pltpu.VMEM((NB, 1, CH), jnp.int32),
            pltpu.VMEM((n_chunks, 1, CH), jnp.int32),
            pltpu.VMEM((NB, CH, d_node), jnp.float32),
            pltpu.VMEM_SHARED((n, d_node), jnp.float32),
            pltpu.SemaphoreType.DMA,
            pltpu.SemaphoreType.DMA,
            pltpu.SemaphoreType.DMA,
            pltpu.SemaphoreType.DMA,
            pltpu.SemaphoreType.DMA,
            pltpu.SemaphoreType.DMA,
            pltpu.SemaphoreType.DMA,
        ],
    )
    def g_kernel(x_hbm, row_hbm, col_hbm, zb_hbm, g_out,
                 idxr_v, cidx2_v, rows_v, g_sh,
                 sg0, sg1, sem_aux, si0, si1, ss0, ss1):
        c = lax.axis_index("c")
        s = lax.axis_index("s")
        sem_g = [sg0, sg1]
        sem_i = [si0, si1]
        sems = [ss0, ss1]
        wid = c * NS + s
        last = n_chunks - 1

        pltpu.async_copy(col_hbm.at[wid], cidx2_v, sem_aux)
        pltpu.async_copy(row_hbm.at[wid, 0], idxr_v.at[0], sem_i[0])
        pltpu.make_async_copy(col_hbm.at[wid], cidx2_v, sem_aux).wait()
        _zero_spmem(s, zb_hbm, g_sh, n, sem_aux)
        plsc.subcore_barrier()

        pltpu.make_async_copy(row_hbm.at[wid, 0], idxr_v.at[0],
                              sem_i[0]).wait()
        pltpu.async_copy(x_hbm.at[idxr_v.at[0, 0]], rows_v.at[0], sem_g[0])
        pltpu.async_copy(row_hbm.at[wid, 1], idxr_v.at[1], sem_i[1])

        def proc(i, b):
            rb = rows_v.at[b]
            civ = cidx2_v.at[i, 0]
            pltpu.make_async_copy(x_hbm.at[idxr_v.at[b, 0]], rb,
                                  sem_g[b]).wait()
            @pl.when(i > 0)
            def _():
                pltpu.make_async_copy(rows_v.at[1 - b],
                                      g_sh.at[cidx2_v.at[i, 0]],
                                      sems[1 - b]).wait()
            pltpu.make_async_copy(row_hbm.at[wid, i], idxr_v.at[1 - b],
                                  sem_i[1 - b]).wait()
            pltpu.async_copy(x_hbm.at[idxr_v.at[1 - b, 0]],
                             rows_v.at[1 - b], sem_g[1 - b])
            nxt = jnp.minimum(i + 2, last)
            pltpu.async_copy(row_hbm.at[wid, nxt], idxr_v.at[b], sem_i[b])
            pltpu.async_copy(rb, g_sh.at[civ], sems[b], add=True)

        def pair(k, carry):
            proc(2 * k, 0)
            proc(2 * k + 1, 1)
            return carry
        assert n_chunks % 2 == 1 and n_chunks >= 3
        lax.fori_loop(0, (n_chunks - 1) // 2, pair, 0)
        i = n_chunks - 1
        rb = rows_v.at[0]
        civ = cidx2_v.at[i, 0]
        pltpu.make_async_copy(x_hbm.at[idxr_v.at[0, 0]], rb, sem_g[0]).wait()
        pltpu.make_async_copy(rows_v.at[1], g_sh.at[civ], sems[1]).wait()
        pltpu.async_copy(rb, g_sh.at[civ], sems[0], add=True)
        pltpu.make_async_copy(rows_v.at[0], g_sh.at[civ], sems[0]).wait()
        pltpu.make_async_copy(row_hbm.at[wid, 0], idxr_v.at[1],
                              sem_i[1]).wait()

        plsc.subcore_barrier()
        _writeback(s, c, g_sh, g_out, n, sem_aux)

    zb = jnp.zeros((CH, d_node), jnp.float32)
    return g_kernel(x, row4, col4, zb)


def _sc_edge_pass(col4, ea_pad, n, e, d_node):
    n_chunks = e // (NW * CH)

    @functools.partial(
        pl.kernel,
        out_type=jax.ShapeDtypeStruct((NC * n, d_node), jnp.float32),
        mesh=_mesh(),
        scratch_types=[
            pltpu.VMEM((n_chunks, 1, CH), jnp.int32),
            pltpu.VMEM((NB, CH, d_node), jnp.float32),
            pltpu.VMEM_SHARED((n, d_node), jnp.float32),
            pltpu.SemaphoreType.DMA,
            pltpu.SemaphoreType.DMA,
            pltpu.SemaphoreType.DMA,
            pltpu.SemaphoreType.DMA,
            pltpu.SemaphoreType.DMA,
        ],
    )
    def e_kernel(col_hbm, eap_hbm, zb_hbm, s_out,
                 cidx2_v, pad_v, s_sh,
                 sl0, sl1, sem_aux, ss0, ss1):
        c = lax.axis_index("c")
        s = lax.axis_index("s")
        sem_l = [sl0, sl1]
        sems = [ss0, ss1]
        wid = c * NS + s
        ebase = wid * n_chunks

        pltpu.async_copy(col_hbm.at[wid], cidx2_v, sem_aux)
        pltpu.make_async_copy(col_hbm.at[wid], cidx2_v, sem_aux).wait()
        _zero_spmem(s, zb_hbm, s_sh, n, sem_aux)
        plsc.subcore_barrier()

        pltpu.async_copy(eap_hbm.at[pl.ds(ebase * CH, CH)], pad_v.at[0],
                         sem_l[0])

        def proc(i, b):
            rb = pad_v.at[b]
            civ = cidx2_v.at[i, 0]
            pltpu.make_async_copy(eap_hbm.at[pl.ds((ebase + i) * CH, CH)],
                                  rb, sem_l[b]).wait()
            @pl.when(i > 0)
            def _():
                pltpu.make_async_copy(pad_v.at[1 - b], s_sh.at[civ],
                                      sems[1 - b]).wait()
            nxt = jnp.minimum(i + 1, n_chunks - 1)
            pltpu.async_copy(eap_hbm.at[pl.ds((ebase + nxt) * CH, CH)],
                             pad_v.at[1 - b], sem_l[1 - b])
            pltpu.async_copy(rb, s_sh.at[civ], sems[b], add=True)

        def pair(k, carry):
            proc(2 * k, 0)
            proc(2 * k + 1, 1)
            return carry
        assert n_chunks % 2 == 1 and n_chunks >= 3
        lax.fori_loop(0, (n_chunks - 1) // 2, pair, 0)
        i = n_chunks - 1
        rb = pad_v.at[0]
        civ = cidx2_v.at[i, 0]
        pltpu.make_async_copy(eap_hbm.at[pl.ds((ebase + i) * CH, CH)],
                              rb, sem_l[0]).wait()
        pltpu.make_async_copy(pad_v.at[1], s_sh.at[civ], sems[1]).wait()
        pltpu.async_copy(rb, s_sh.at[civ], sems[0], add=True)
        pltpu.make_async_copy(pad_v.at[0], s_sh.at[civ], sems[0]).wait()

        plsc.subcore_barrier()
        _writeback(s, c, s_sh, s_out, n, sem_aux)

    zb = jnp.zeros((CH, d_node), jnp.float32)
    return e_kernel(col4, ea_pad, zb)


def _tc_dense(x, gp, sp, W1, b1, W2, b2, n, d_node, d_edge):
    def body(x_ref, gp_ref, sp_ref, w1_ref, b1_ref, w2_ref, b2_ref,
             o_ref):
        g = gp_ref[:n, :] + gp_ref[n:, :]
        sd = sp_ref[:n, :] + sp_ref[n:, :]
        s_ = sd[:, :d_edge]
        deg = sd[:, d_edge:d_edge + 1]
        W1x = w1_ref[:, :d_node]
        W1e = w1_ref[:, d_node:]
        W2x = w2_ref[:, :d_node]
        W2a = w2_ref[:, d_node:]
        dn = (((1,), (1,)), ((), ()))
        agg = (lax.dot_general(g, W1x, dn, preferred_element_type=jnp.float32)
               + lax.dot_general(s_, W1e, dn, preferred_element_type=jnp.float32)
               + deg * b1_ref[0, :][None, :])
        out = (lax.dot_general(x_ref[...], W2x, dn,
                               preferred_element_type=jnp.float32)
               + lax.dot_general(agg, W2a, dn,
                                 preferred_element_type=jnp.float32)
               + b2_ref[0, :][None, :])
        o_ref[...] = out

    return pl.pallas_call(
        body,
        out_shape=jax.ShapeDtypeStruct((n, d_node), jnp.float32),
    )(x, gp, sp, W1, b1.reshape(1, -1), W2, b2.reshape(1, -1))


def kernel(x, edge_index, edge_attr, u, batch, W1, b1, W2, b2):
    n, d_node = x.shape
    e, d_edge = edge_attr.shape
    n_chunks = e // (NW * CH)
    row4 = edge_index[0].reshape(NW, n_chunks, 1, CH)
    col4 = edge_index[1].reshape(NW, n_chunks, 1, CH)
    ea_pad = jnp.concatenate(
        [edge_attr, jnp.ones((e, 1), jnp.float32),
         jnp.zeros((e, d_node - d_edge - 1), jnp.float32)], axis=1)
    gp = _sc_gather_pass(x, row4, col4, n, e, d_node)
    sp = _sc_edge_pass(col4, ea_pad, n, e, d_node)
    return _tc_dense(x, gp, sp, W1, b1, W2, b2, n, d_node, d_edge)


# speedup vs baseline: 1.1267x; 1.1267x over previous
import functools

import jax
import jax.numpy as jnp
from jax import lax
from jax.experimental import pallas as pl
from jax.experimental.pallas import tpu as pltpu
from jax.experimental.pallas import tpu_sc as plsc

NC = 2
NS = 16
NW = NC * NS
CH = 128
ZCH = 80
NB = 2
NI = 3


def _mesh():
    return plsc.VectorSubcoreMesh(
        core_axis_name="c", subcore_axis_name="s", num_cores=NC,
        num_subcores=NS)


def _zero_spmem(s, zb_hbm, acc_sh, n, sem):
    n_z = n // ZCH
    n_iter = (n_z + NS - 1) // NS
    def fire(j, carry):
        k = s + j * NS
        @pl.when(k < n_z)
        def _():
            pltpu.async_copy(zb_hbm, acc_sh.at[pl.ds(k * ZCH, ZCH)], sem)
        return carry
    lax.fori_loop(0, n_iter, fire, 0)
    def drain(j, carry):
        k = s + j * NS
        @pl.when(k < n_z)
        def _():
            pltpu.make_async_copy(zb_hbm, acc_sh.at[pl.ds(k * ZCH, ZCH)],
                                  sem).wait()
        return carry
    lax.fori_loop(0, n_iter, drain, 0)


def _writeback(s, c, acc_sh, out_hbm, n, sem):
    n_z = n // ZCH
    n_iter = (n_z + NS - 1) // NS
    def fire(j, carry):
        k = s + j * NS
        @pl.when(k < n_z)
        def _():
            pltpu.async_copy(acc_sh.at[pl.ds(k * ZCH, ZCH)],
                             out_hbm.at[pl.ds(c * n + k * ZCH, ZCH)], sem)
        return carry
    lax.fori_loop(0, n_iter, fire, 0)
    def drain(j, carry):
        k = s + j * NS
        @pl.when(k < n_z)
        def _():
            pltpu.make_async_copy(acc_sh.at[pl.ds(k * ZCH, ZCH)],
                                  out_hbm.at[pl.ds(c * n + k * ZCH, ZCH)],
                                  sem).wait()
        return carry
    lax.fori_loop(0, n_iter, drain, 0)


def _plan(e):
    n_gchunks = e // CH
    base = n_gchunks // NW
    xtra = n_gchunks - base * NW
    assert e == n_gchunks * CH and base % 6 == 0 and base >= 6
    return n_gchunks, base, xtra


def _sc_gather_pass(x, idx_cat, n, e, d_node):
    n_gchunks, base, xtra = _plan(e)

    @functools.partial(
        pl.kernel,
        out_type=jax.ShapeDtypeStruct((NC * n, d_node), jnp.float32),
        mesh=_mesh(),
        scratch_types=[
            pltpu.VMEM((NI, 2, CH), jnp.int32),
            pltpu.VMEM((NB, CH, d_node), jnp.float32),
            pltpu.VMEM_SHARED((n, d_node), jnp.float32),
            pltpu.SemaphoreType.DMA,
            pltpu.SemaphoreType.DMA,
            pltpu.SemaphoreType.DMA,
            pltpu.SemaphoreType.DMA,
            pltpu.SemaphoreType.DMA,
            pltpu.SemaphoreType.DMA,
            pltpu.SemaphoreType.DMA,
        ],
    )
    def g_kernel(x_hbm, idx_hbm, zb_hbm, g_out,
                 idxr_v, rows_v, g_sh,
                 sem_g, sem_aux, si0, si1, si2, ss0, ss1):
        c = lax.axis_index("c")
        s = lax.axis_index("s")
        sem_i = [si0, si1, si2]
        sems = [ss0, ss1]
        wid = c * NS + s
        gmax = n_gchunks - 1

        pltpu.async_copy(idx_hbm.at[wid], idxr_v.at[0], sem_i[0])
        _zero_spmem(s, zb_hbm, g_sh, n, sem_aux)
        plsc.subcore_barrier()

        def chunk(i, b2, b3, first):
            rb = rows_v.at[b2]
            riv = idxr_v.at[b3, 0]
            civ = idxr_v.at[b3, 1]
            if not first:
                pltpu.make_async_copy(rb, g_sh.at[civ], sems[b2]).wait()
            pltpu.make_async_copy(idx_hbm.at[wid], idxr_v.at[b3],
                                  sem_i[b3]).wait()
            pltpu.async_copy(x_hbm.at[riv], rb, sem_g)
            nxt = jnp.minimum(wid + (i + 1) * NW, gmax)
            pltpu.async_copy(idx_hbm.at[nxt], idxr_v.at[(0 if b3 == NI - 1
                                                         else b3 + 1)],
                             sem_i[(0 if b3 == NI - 1 else b3 + 1)])
            pltpu.make_async_copy(x_hbm.at[riv], rb, sem_g).wait()
            pltpu.async_copy(rb, g_sh.at[civ], sems[b2], add=True)

        def six(k, carry):
            for j in range(6):
                i = 6 * k + j
                if j < 2:
                    @pl.when(k > 0)
                    def _(i=i, j=j):
                        chunk(i, j % NB, j % NI, False)
                    @pl.when(k == 0)
                    def _(i=i, j=j):
                        chunk(i, j % NB, j % NI, True)
                else:
                    chunk(i, j % NB, j % NI, False)
            return carry
        lax.fori_loop(0, base // 6, six, 0)
        for b in range(NB):
            pltpu.make_async_copy(rows_v.at[b], g_sh.at[idxr_v.at[0, 1]],
                                  sems[b]).wait()
        @pl.when(wid < xtra)
        def _():
            rb = rows_v.at[0]
            riv = idxr_v.at[0, 0]
            civ = idxr_v.at[0, 1]
            pltpu.make_async_copy(idx_hbm.at[wid], idxr_v.at[0],
                                  sem_i[0]).wait()
            pltpu.async_copy(x_hbm.at[riv], rb, sem_g).wait()
            pltpu.async_copy(rb, g_sh.at[civ], sems[0], add=True)
            pltpu.make_async_copy(rb, g_sh.at[civ], sems[0]).wait()
        @pl.when(wid >= xtra)
        def _():
            pltpu.make_async_copy(idx_hbm.at[wid], idxr_v.at[0],
                                  sem_i[0]).wait()

        plsc.subcore_barrier()
        _writeback(s, c, g_sh, g_out, n, sem_aux)

    zb = jnp.zeros((ZCH, d_node), jnp.float32)
    return g_kernel(x, idx_cat, zb)


def _sc_edge_pass(idx_cat, ea_pad, n, e, d_node):
    n_gchunks, base, xtra = _plan(e)

    @functools.partial(
        pl.kernel,
        out_type=jax.ShapeDtypeStruct((NC * n, d_node), jnp.float32),
        mesh=_mesh(),
        scratch_types=[
            pltpu.VMEM((NI, 2, CH), jnp.int32),
            pltpu.VMEM((NB, CH, d_node), jnp.float32),
            pltpu.VMEM_SHARED((n, d_node), jnp.float32),
            pltpu.SemaphoreType.DMA,
            pltpu.SemaphoreType.DMA,
            pltpu.SemaphoreType.DMA,
            pltpu.SemaphoreType.DMA,
            pltpu.SemaphoreType.DMA,
            pltpu.SemaphoreType.DMA,
            pltpu.SemaphoreType.DMA,
        ],
    )
    def e_kernel(idx_hbm, eap_hbm, zb_hbm, s_out,
                 idxr_v, pad_v, s_sh,
                 sem_l, sem_aux, si0, si1, si2, ss0, ss1):
        c = lax.axis_index("c")
        s = lax.axis_index("s")
        sem_i = [si0, si1, si2]
        sems = [ss0, ss1]
        wid = c * NS + s
        gmax = n_gchunks - 1

        pltpu.async_copy(idx_hbm.at[wid], idxr_v.at[0], sem_i[0])
        _zero_spmem(s, zb_hbm, s_sh, n, sem_aux)
        plsc.subcore_barrier()

        def chunk(i, b2, b3, first):
            gc = wid + i * NW
            rb = pad_v.at[b2]
            civ = idxr_v.at[b3, 1]
            if not first:
                pltpu.make_async_copy(rb, s_sh.at[civ], sems[b2]).wait()
            pltpu.make_async_copy(idx_hbm.at[wid], idxr_v.at[b3],
                                  sem_i[b3]).wait()
            pltpu.async_copy(eap_hbm.at[pl.ds(gc * CH, CH)], rb, sem_l)
            nxt = jnp.minimum(wid + (i + 1) * NW, gmax)
            pltpu.async_copy(idx_hbm.at[nxt], idxr_v.at[(0 if b3 == NI - 1
                                                         else b3 + 1)],
                             sem_i[(0 if b3 == NI - 1 else b3 + 1)])
            pltpu.make_async_copy(eap_hbm.at[pl.ds(gc * CH, CH)], rb,
                                  sem_l).wait()
            pltpu.async_copy(rb, s_sh.at[civ], sems[b2], add=True)

        def six(k, carry):
            for j in range(6):
                i = 6 * k + j
                if j < 2:
                    @pl.when(k > 0)
                    def _(i=i, j=j):
                        chunk(i, j % NB, j % NI, False)
                    @pl.when(k == 0)
                    def _(i=i, j=j):
                        chunk(i, j % NB, j % NI, True)
                else:
                    chunk(i, j % NB, j % NI, False)
            return carry
        lax.fori_loop(0, base // 6, six, 0)
        for b in range(NB):
            pltpu.make_async_copy(pad_v.at[b], s_sh.at[idxr_v.at[0, 1]],
                                  sems[b]).wait()
        @pl.when(wid < xtra)
        def _():
            gc = wid + base * NW
            rb = pad_v.at[0]
            civ = idxr_v.at[0, 1]
            pltpu.make_async_copy(idx_hbm.at[wid], idxr_v.at[0],
                                  sem_i[0]).wait()
            pltpu.async_copy(eap_hbm.at[pl.ds(gc * CH, CH)], rb,
                             sem_l).wait()
            pltpu.async_copy(rb, s_sh.at[civ], sems[0], add=True)
            pltpu.make_async_copy(rb, s_sh.at[civ], sems[0]).wait()
        @pl.when(wid >= xtra)
        def _():
            pltpu.make_async_copy(idx_hbm.at[wid], idxr_v.at[0],
                                  sem_i[0]).wait()

        plsc.subcore_barrier()
        _writeback(s, c, s_sh, s_out, n, sem_aux)

    zb = jnp.zeros((ZCH, d_node), jnp.float32)
    return e_kernel(idx_cat, ea_pad, zb)


def _tc_dense(x, gp, sp, W1, b1, W2, b2, n, d_node, d_edge):
    def body(x_ref, gp_ref, sp_ref, w1_ref, b1_ref, w2_ref, b2_ref,
             o_ref):
        g = gp_ref[:n, :] + gp_ref[n:, :]
        sd = sp_ref[:n, :] + sp_ref[n:, :]
        s_ = sd[:, :d_edge]
        deg = sd[:, d_edge:d_edge + 1]
        W1x = w1_ref[:, :d_node]
        W1e = w1_ref[:, d_node:]
        W2x = w2_ref[:, :d_node]
        W2a = w2_ref[:, d_node:]
        dn = (((1,), (1,)), ((), ()))
        agg = (lax.dot_general(g, W1x, dn, preferred_element_type=jnp.float32)
               + lax.dot_general(s_, W1e, dn, preferred_element_type=jnp.float32)
               + deg * b1_ref[0, :][None, :])
        out = (lax.dot_general(x_ref[...], W2x, dn,
                               preferred_element_type=jnp.float32)
               + lax.dot_general(agg, W2a, dn,
                                 preferred_element_type=jnp.float32)
               + b2_ref[0, :][None, :])
        o_ref[...] = out

    return pl.pallas_call(
        body,
        out_shape=jax.ShapeDtypeStruct((n, d_node), jnp.float32),
    )(x, gp, sp, W1, b1.reshape(1, -1), W2, b2.reshape(1, -1))


def kernel(x, edge_index, edge_attr, u, batch, W1, b1, W2, b2):
    n, d_node = x.shape
    e, d_edge = edge_attr.shape
    idx_cat = jnp.stack(
        [edge_index[0].reshape(e // CH, CH),
         edge_index[1].reshape(e // CH, CH)], axis=1)
    ea_pad = jnp.concatenate(
        [edge_attr, jnp.ones((e, 1), jnp.float32),
         jnp.zeros((e, d_node - d_edge - 1), jnp.float32)], axis=1)
    gp = _sc_gather_pass(x, idx_cat, n, e, d_node)
    sp = _sc_edge_pass(idx_cat, ea_pad, n, e, d_node)
    return _tc_dense(x, gp, sp, W1, b1, W2, b2, n, d_node, d_edge)
